# Initial kernel scaffold; baseline (speedup 1.0000x reference)
import jax
import jax.numpy as jnp
from jax.experimental import pallas as pl


def kernel(input_ids, embedding):
    raise NotImplementedError("write your pallas kernel here")


# speedup vs baseline: 2.0381x; 2.0381x over previous
import functools

import jax
import jax.numpy as jnp
from jax import lax
from jax.experimental import pallas as pl
from jax.experimental.pallas import tpu as pltpu
from jax.experimental.pallas import tpu_sc as plsc

ROWS, SEQ, D = 16384, 200, 128
TOTAL = ROWS * SEQ
NC, NS = 2, 16
NW = NC * NS
PER_W = TOTAL // NW
CHUNK = 128
NBUF = 4
SUPER = 25600
N_SUPER = PER_W // SUPER
CPS = SUPER // CHUNK

assert N_SUPER * SUPER == PER_W and CPS * CHUNK == SUPER and CPS % NBUF == 0

_mesh = plsc.VectorSubcoreMesh(core_axis_name="c", subcore_axis_name="s")


@functools.partial(
    pl.kernel,
    out_type=jax.ShapeDtypeStruct((TOTAL, D), jnp.float32),
    mesh=_mesh,
    scratch_types=[
        pltpu.VMEM((SUPER,), jnp.int32),
        pltpu.VMEM((NBUF, CHUNK, D), jnp.float32),
        pltpu.SemaphoreType.DMA,
        pltpu.SemaphoreType.DMA,
    ],
)
def _emb_lookup(ids_hbm, table_hbm, out_hbm, ids_v, rows_v, gsem, osem):
    wid = lax.axis_index("s") * NC + lax.axis_index("c")
    base = wid * PER_W

    @pl.loop(0, N_SUPER)
    def _supers(s):
        sbase = base + s * SUPER
        pltpu.sync_copy(ids_hbm.at[pl.ds(sbase, SUPER)], ids_v)

        @pl.loop(0, CPS, step=NBUF)
        def _chunks(i0):
            for b in range(NBUF):
                i = i0 + b
                off = sbase + i * CHUNK

                @pl.when(i0 >= NBUF)
                def _drain_prev():
                    pltpu.make_async_copy(
                        rows_v.at[b],
                        out_hbm.at[pl.ds(off - NBUF * CHUNK, CHUNK)],
                        osem,
                    ).wait()

                pltpu.async_copy(
                    table_hbm.at[ids_v.at[pl.ds(i * CHUNK, CHUNK)]],
                    rows_v.at[b],
                    gsem,
                ).wait()
                pltpu.async_copy(
                    rows_v.at[b],
                    out_hbm.at[pl.ds(off, CHUNK)],
                    osem,
                )

        for b in range(NBUF):
            i = CPS - NBUF + b
            pltpu.make_async_copy(
                rows_v.at[b],
                out_hbm.at[pl.ds(sbase + i * CHUNK, CHUNK)],
                osem,
            ).wait()


def kernel(input_ids, embedding):
    ids_flat = input_ids.reshape(-1).astype(jnp.int32)
    out = _emb_lookup(ids_flat, embedding)
    return out.reshape(ROWS, SEQ, D)


# speedup vs baseline: 18.1743x; 8.9172x over previous
import functools

import jax
import jax.numpy as jnp
from jax import lax
from jax.experimental import pallas as pl
from jax.experimental.pallas import tpu as pltpu
from jax.experimental.pallas import tpu_sc as plsc

ROWS, SEQ, D = 16384, 200, 128
TOTAL = ROWS * SEQ
NC, NS = 2, 16
NW = NC * NS
PER_W = TOTAL // NW
CHUNK = 128
NBUF = 4
SUPER = 25600
N_SUPER = PER_W // SUPER
CPS = SUPER // CHUNK

assert N_SUPER * SUPER == PER_W and CPS * CHUNK == SUPER and CPS % NBUF == 0

_mesh = plsc.VectorSubcoreMesh(core_axis_name="c", subcore_axis_name="s")


@functools.partial(
    pl.kernel,
    out_type=jax.ShapeDtypeStruct((TOTAL, D), jnp.float32),
    mesh=_mesh,
    scratch_types=[
        pltpu.VMEM((SUPER,), jnp.int32),
        pltpu.VMEM((NBUF, CHUNK, D), jnp.float32),
        pltpu.VMEM_SHARED((32, D), jnp.float32),
        pltpu.SemaphoreType.DMA,
        pltpu.SemaphoreType.DMA,
    ],
)
def _emb_lookup(ids_hbm, table_hbm, out_hbm, ids_v, rows_v, tab_sh, gsem, osem):
    wid = lax.axis_index("s") * NC + lax.axis_index("c")
    base = wid * PER_W

    @pl.when(lax.axis_index("s") == 0)
    def _stage_table():
        pltpu.sync_copy(table_hbm, tab_sh)

    plsc.subcore_barrier()

    @pl.loop(0, N_SUPER)
    def _supers(s):
        sbase = base + s * SUPER
        pltpu.sync_copy(ids_hbm.at[pl.ds(sbase, SUPER)], ids_v)

        @pl.loop(0, CPS, step=NBUF)
        def _chunks(i0):
            for b in range(NBUF):
                i = i0 + b
                off = sbase + i * CHUNK

                @pl.when(i0 >= NBUF)
                def _drain_prev():
                    pltpu.make_async_copy(
                        rows_v.at[b],
                        out_hbm.at[pl.ds(off - NBUF * CHUNK, CHUNK)],
                        osem,
                    ).wait()

                pltpu.async_copy(
                    tab_sh.at[ids_v.at[pl.ds(i * CHUNK, CHUNK)]],
                    rows_v.at[b],
                    gsem,
                ).wait()
                pltpu.async_copy(
                    rows_v.at[b],
                    out_hbm.at[pl.ds(off, CHUNK)],
                    osem,
                )

        for b in range(NBUF):
            i = CPS - NBUF + b
            pltpu.make_async_copy(
                rows_v.at[b],
                out_hbm.at[pl.ds(sbase + i * CHUNK, CHUNK)],
                osem,
            ).wait()


def kernel(input_ids, embedding):
    ids_flat = input_ids.reshape(-1).astype(jnp.int32)
    out = _emb_lookup(ids_flat, embedding)
    return out.reshape(ROWS, SEQ, D)
